# baseline (speedup 1.0000x reference)
import functools

import jax
import jax.numpy as jnp
from jax import lax
from jax.experimental import pallas as pl
from jax.experimental.pallas import tpu as pltpu
from jax.experimental.pallas import tpu_sc as plsc

_NUM_CORES = 2
_NUM_SUBCORES = 16
_NW = _NUM_CORES * _NUM_SUBCORES


@functools.lru_cache(maxsize=None)
def _build(B, D, chunk):
    b_per_w = B // _NW
    steps = b_per_w // chunk
    mesh = plsc.VectorSubcoreMesh(core_axis_name="c", subcore_axis_name="s")

    @functools.partial(
        pl.kernel,
        mesh=mesh,
        out_type=jax.ShapeDtypeStruct((B, D), jnp.float32),
        compiler_params=pltpu.CompilerParams(use_tc_tiling_on_sc=False),
        scratch_types=[
            pltpu.VMEM((chunk,), jnp.int32),
            pltpu.VMEM((chunk, D), jnp.float32),
            pltpu.SemaphoreType.DMA,
        ],
    )
    def k(idx_hbm, table_hbm, out_hbm, idx_v, rows_v, sem):
        wid = lax.axis_index("s") * _NUM_CORES + lax.axis_index("c")
        base = wid * b_per_w

        def body(i, carry):
            off = base + i * chunk
            pltpu.sync_copy(idx_hbm.at[pl.ds(off, chunk)], idx_v)
            pltpu.async_copy(table_hbm.at[idx_v], rows_v, sem).wait()
            pltpu.sync_copy(rows_v, out_hbm.at[pl.ds(off, chunk)])
            return carry

        lax.fori_loop(0, steps, body, 0)

    return k


def kernel(x, table):
    B0, S = x.shape
    V, D = table.shape
    B = B0 * S
    idx = x.reshape(B)
    out = _build(B, D, 128)(idx, table)
    return out.reshape(B0, S, D)


# speedup vs baseline: 1.1113x; 1.1113x over previous
import functools

import jax
import jax.numpy as jnp
from jax import lax
from jax.experimental import pallas as pl
from jax.experimental.pallas import tpu as pltpu
from jax.experimental.pallas import tpu_sc as plsc

_NUM_CORES = 2
_NUM_SUBCORES = 16
_NW = _NUM_CORES * _NUM_SUBCORES


@functools.lru_cache(maxsize=None)
def _build(B, D, chunk):
    b_per_w = B // _NW
    steps = b_per_w // chunk
    assert steps % 2 == 0 and steps >= 4
    mesh = plsc.VectorSubcoreMesh(core_axis_name="c", subcore_axis_name="s")

    @functools.partial(
        pl.kernel,
        mesh=mesh,
        out_type=jax.ShapeDtypeStruct((B, D), jnp.float32),
        compiler_params=pltpu.CompilerParams(use_tc_tiling_on_sc=False),
        scratch_types=[
            pltpu.VMEM((steps, chunk), jnp.int32),
            pltpu.VMEM((2, chunk, D), jnp.float32),
            pltpu.SemaphoreType.DMA,
            pltpu.SemaphoreType.DMA,
            pltpu.SemaphoreType.DMA,
            pltpu.SemaphoreType.DMA,
        ],
    )
    def k(idx_hbm, table_hbm, out_hbm, idx_v, rows_v, g0, g1, o0, o1):
        gsem = (g0, g1)
        osem = (o0, o1)
        wid = lax.axis_index("s") * _NUM_CORES + lax.axis_index("c")
        base = wid * b_per_w
        pltpu.sync_copy(idx_hbm.at[wid], idx_v)

        def gather_start(i, b):
            pltpu.async_copy(table_hbm.at[idx_v.at[i]], rows_v.at[b], gsem[b])

        def gather_wait(i, b):
            pltpu.make_async_copy(
                table_hbm.at[idx_v.at[i]], rows_v.at[b], gsem[b]).wait()

        def store_start(i, b):
            pltpu.async_copy(
                rows_v.at[b], out_hbm.at[pl.ds(base + i * chunk, chunk)],
                osem[b])

        def store_wait(i, b):
            pltpu.make_async_copy(
                rows_v.at[b], out_hbm.at[pl.ds(base + i * chunk, chunk)],
                osem[b]).wait()

        gather_start(0, 0)

        def body(j, carry):
            for b in range(2):
                i = j * 2 + b
                ob = 1 - b
                gather_wait(i, b)

                @pl.when(i + 1 < steps)
                def _():
                    @pl.when(i >= 1)
                    def _():
                        store_wait(i - 1, ob)

                    gather_start(i + 1, ob)

                store_start(i, b)
            return carry

        lax.fori_loop(0, steps // 2, body, 0)
        store_wait(steps - 2, 0)
        store_wait(steps - 1, 1)

    return k


def kernel(x, table):
    B0, S = x.shape
    V, D = table.shape
    B = B0 * S
    chunk = 128
    b_per_w = B // _NW
    idx = x.reshape(_NW, b_per_w // chunk, chunk)
    out = _build(B, D, chunk)(idx, table)
    return out.reshape(B0, S, D)


# speedup vs baseline: 1.1908x; 1.0716x over previous
import functools

import jax
import jax.numpy as jnp
from jax import lax
from jax.experimental import pallas as pl
from jax.experimental.pallas import tpu as pltpu
from jax.experimental.pallas import tpu_sc as plsc

_NUM_CORES = 2
_NUM_SUBCORES = 16
_NW = _NUM_CORES * _NUM_SUBCORES


@functools.lru_cache(maxsize=None)
def _build(B, D, chunk):
    b_per_w = B // _NW
    steps = b_per_w // chunk
    assert steps % 2 == 0 and steps >= 4
    mesh = plsc.VectorSubcoreMesh(core_axis_name="c", subcore_axis_name="s")

    @functools.partial(
        pl.kernel,
        mesh=mesh,
        out_type=jax.ShapeDtypeStruct((B, D), jnp.float32),
        compiler_params=pltpu.CompilerParams(use_tc_tiling_on_sc=False),
        scratch_types=[
            pltpu.VMEM((steps, chunk), jnp.int32),
            pltpu.VMEM((2, chunk, D), jnp.float32),
            pltpu.SemaphoreType.DMA,
            pltpu.SemaphoreType.DMA,
            pltpu.SemaphoreType.DMA,
            pltpu.SemaphoreType.DMA,
        ],
    )
    def k(idx_hbm, table_hbm, out_hbm, idx_v, rows_v, g0, g1, o0, o1):
        gsem = (g0, g1)
        osem = (o0, o1)
        wid = lax.axis_index("s") * _NUM_CORES + lax.axis_index("c")
        base = wid * b_per_w
        pltpu.sync_copy(idx_hbm.at[wid], idx_v)

        def gather_start(i, b):
            pltpu.async_copy(table_hbm.at[idx_v.at[i]], rows_v.at[b], gsem[b])

        def gather_wait(i, b):
            pltpu.make_async_copy(
                table_hbm.at[idx_v.at[i]], rows_v.at[b], gsem[b]).wait()

        def store_start(i, b):
            pltpu.async_copy(
                rows_v.at[b], out_hbm.at[pl.ds(base + i * chunk, chunk)],
                osem[b])

        def store_wait(i, b):
            pltpu.make_async_copy(
                rows_v.at[b], out_hbm.at[pl.ds(base + i * chunk, chunk)],
                osem[b]).wait()

        gather_start(0, 0)

        def body(j, carry):
            for b in range(2):
                i = j * 2 + b
                ob = 1 - b
                gather_wait(i, b)

                @pl.when(i + 1 < steps)
                def _():
                    @pl.when(i >= 1)
                    def _():
                        store_wait(i - 1, ob)

                    gather_start(i + 1, ob)

                store_start(i, b)
            return carry

        lax.fori_loop(0, steps // 2, body, 0)
        store_wait(steps - 2, 0)
        store_wait(steps - 1, 1)

    return k


def kernel(x, table):
    B0, S = x.shape
    V, D = table.shape
    B = B0 * S
    chunk = 512
    b_per_w = B // _NW
    idx = x.reshape(_NW, b_per_w // chunk, chunk)
    out = _build(B, D, chunk)(idx, table)
    return out.reshape(B0, S, D)
